# baseline (speedup 1.0000x reference)
import numpy as np
import jax
import jax.numpy as jnp
from jax import lax
from jax.experimental import pallas as pl

_N = 6402
_NP = 6528
_C = 80
_K = 200
_M = 100
_FEATS = [40, 20, 10, 5, 3]
_STRIDES = [8, 16, 32, 64, 128]
_MAXR = 4.135166556742356


def _anchors_const():
    ratios = np.array([0.5, 1.0, 2.0], dtype=np.float64)
    h_r = np.sqrt(ratios)
    w_r = 1.0 / h_r
    out = []
    for bs, f, st in zip(_STRIDES, _FEATS, _STRIDES):
        cx = bs / 2.0
        cy = bs / 2.0
        ws = (bs * w_r * 3.0).astype(np.float32)
        hs = (bs * h_r * 3.0).astype(np.float32)
        base = np.stack([cx - 0.5 * ws, cy - 0.5 * hs,
                         cx + 0.5 * ws, cy + 0.5 * hs], axis=-1)
        sx = np.arange(f, dtype=np.float32) * st
        xx = np.tile(sx, f)
        yy = np.repeat(sx, f)
        shifts = np.stack([xx, yy, xx, yy], axis=-1)
        out.append((base[None, :, :] + shifts[:, None, :]).reshape(-1, 4))
    a = np.concatenate(out, axis=0).astype(np.float32)
    a = np.pad(a, ((0, _NP - _N), (0, 0)))
    return a.T.copy()


_ANCHORS = _anchors_const()


def _yolact_kernel(lg_ref, dl_ref, cf_ref, an_ref, pr_ref,
                   dets_ref, cls_ref, masks_ref):
    f32 = jnp.float32
    i32 = jnp.int32
    L = lg_ref[...]
    D = dl_ref[...]
    CO = cf_ref[...]
    A = an_ref[...]

    mx = jnp.max(L, axis=0, keepdims=True)
    e = jnp.exp(L - mx)
    p = e / jnp.sum(e, axis=0, keepdims=True)
    colN = lax.broadcasted_iota(i32, (1, _NP), 1)
    S = jnp.where(colN < _N, p[:_C, :], -1.0)

    ax1, ay1, ax2, ay2 = A[0:1], A[1:2], A[2:3], A[3:4]
    dx = D[0:1] * 0.1
    dy = D[1:2] * 0.1
    dw = jnp.clip(D[2:3] * 0.2, -_MAXR, _MAXR)
    dh = jnp.clip(D[3:4] * 0.2, -_MAXR, _MAXR)
    px = (ax1 + ax2) * 0.5
    py = (ay1 + ay2) * 0.5
    pw = ax2 - ax1
    ph = ay2 - ay1
    gx = px + pw * dx
    gy = py + ph * dy
    gw = pw * jnp.exp(dw)
    gh = ph * jnp.exp(dh)
    X1 = jnp.clip(gx - gw * 0.5, 0.0, 320.0)
    Y1 = jnp.clip(gy - gh * 0.5, 0.0, 320.0)
    X2 = jnp.clip(gx + gw * 0.5, 0.0, 320.0)
    Y2 = jnp.clip(gy + gh * 0.5, 0.0, 320.0)

    colCN = lax.broadcasted_iota(i32, (_C, _NP), 1)
    lane200 = lax.broadcasted_iota(i32, (1, _K), 1)

    def bcond(carry):
        return (carry[0] < _K) & (carry[-1] > 0.05)

    def bstep(carry):
        t, S_, tS, tI, TX1, TY1, TX2, TY2, _ = carry
        m = jnp.max(S_, axis=1, keepdims=True)
        am = jnp.min(jnp.where(S_ == m, colCN, _NP), axis=1,
                     keepdims=True)
        cm = colCN == am
        bx1 = jnp.sum(jnp.where(cm, X1, 0.0), axis=1, keepdims=True)
        by1 = jnp.sum(jnp.where(cm, Y1, 0.0), axis=1, keepdims=True)
        bx2 = jnp.sum(jnp.where(cm, X2, 0.0), axis=1, keepdims=True)
        by2 = jnp.sum(jnp.where(cm, Y2, 0.0), axis=1, keepdims=True)
        sel = lane200 == t
        tS = jnp.where(sel, m, tS)
        tI = jnp.where(sel, am, tI)
        TX1 = jnp.where(sel, bx1, TX1)
        TY1 = jnp.where(sel, by1, TY1)
        TX2 = jnp.where(sel, bx2, TX2)
        TY2 = jnp.where(sel, by2, TY2)
        S_ = jnp.where(cm, -2.0, S_)
        return t + 1, S_, tS, tI, TX1, TY1, TX2, TY2, jnp.max(m)

    zf = jnp.zeros((_C, _K), f32)
    _, _, topS, topI, TX1, TY1, TX2, TY2, _ = lax.while_loop(
        bcond, bstep,
        (jnp.int32(0), S, zf, jnp.zeros((_C, _K), i32),
         zf, zf, zf, zf, jnp.float32(1.0)))

    rlt = (lax.broadcasted_iota(i32, (_K, _K), 0) <
           lax.broadcasted_iota(i32, (_K, _K), 1))
    rows80 = lax.broadcasted_iota(i32, (_C, 1), 0)

    def dstep(c, Mc):
        rsel = rows80 == c
        x1t = jnp.sum(jnp.where(rsel, TX1, 0.0), axis=0, keepdims=True)
        y1t = jnp.sum(jnp.where(rsel, TY1, 0.0), axis=0, keepdims=True)
        x2t = jnp.sum(jnp.where(rsel, TX2, 0.0), axis=0, keepdims=True)
        y2t = jnp.sum(jnp.where(rsel, TY2, 0.0), axis=0, keepdims=True)
        x1 = x1t.reshape(_K, 1)
        y1 = y1t.reshape(_K, 1)
        x2 = x2t.reshape(_K, 1)
        y2 = y2t.reshape(_K, 1)
        w = jnp.maximum(jnp.minimum(x2, x2t) - jnp.maximum(x1, x1t), 0.0)
        h = jnp.maximum(jnp.minimum(y2, y2t) - jnp.maximum(y1, y1t), 0.0)
        ov = w * h
        ar = (x2 - x1) * (y2 - y1)
        un = jnp.maximum(ar + ar.reshape(1, _K) - ov, 1e-6)
        iou = ov / un
        im = jnp.max(jnp.where(rlt, iou, 0.0), axis=0, keepdims=True)
        s_c = jnp.sum(jnp.where(rsel, topS, 0.0), axis=0, keepdims=True)
        keep = (im <= 0.5) & (s_c > 0.05)
        mk = jnp.where(keep, s_c, -1.0)
        return jnp.where(rows80 == c, mk, Mc)

    Mmat = lax.fori_loop(0, _C, dstep, jnp.full((_C, _K), -1.0, f32))

    flatKC = (lax.broadcasted_iota(i32, (_C, _K), 0) * _K +
              lax.broadcasted_iota(i32, (_C, _K), 1))
    lane128 = lax.broadcasted_iota(i32, (1, 128), 1)
    row8 = lax.broadcasted_iota(i32, (8, 1), 0)

    def estep(k, carry):
        dets, clsv, CF, Mc = carry
        m = jnp.max(Mc)
        fl = jnp.min(jnp.where(Mc == m, flatKC, 1 << 30))
        cc = fl // _K
        anc = jnp.sum(jnp.where(flatKC == fl, topI, 0))
        valid = m > 0.0
        sc = jnp.where(valid, m, 0.0)
        amask = (colN == anc) & valid
        amf = amask.astype(f32)
        bx1 = jnp.sum(amf * X1)
        by1 = jnp.sum(amf * Y1)
        bx2 = jnp.sum(amf * X2)
        by2 = jnp.sum(amf * Y2)
        ck = jnp.sum(jnp.where(amask, CO, 0.0), axis=1, keepdims=True)
        sel = lane128 == k
        dets = jnp.where(sel & (row8 == 0), bx1, dets)
        dets = jnp.where(sel & (row8 == 1), by1, dets)
        dets = jnp.where(sel & (row8 == 2), bx2, dets)
        dets = jnp.where(sel & (row8 == 3), by2, dets)
        dets = jnp.where(sel & (row8 == 4), sc, dets)
        clsv = jnp.where(sel & (row8 == 0), cc, clsv)
        CF = jnp.where(sel, ck, CF)
        Mc = jnp.where(flatKC == fl, -2.0, Mc)
        return dets, clsv, CF, Mc

    dets, clsv, CF, _ = lax.fori_loop(
        0, _M, estep,
        (jnp.zeros((8, 128), f32), jnp.zeros((8, 128), i32),
         jnp.zeros((32, 128), f32), Mmat))

    P = pr_ref[...]
    mp = jnp.dot(P, CF, preferred_element_type=f32)
    sg = jax.nn.sigmoid(mp)
    pix = lax.broadcasted_iota(i32, (6400, 1), 0)
    rr = (pix // 80).astype(f32)
    wc = (pix % 80).astype(f32)
    cx1 = dets[0:1] * 0.25
    cy1 = dets[1:2] * 0.25
    cx2 = dets[2:3] * 0.25
    cy2 = dets[3:4] * 0.25
    cm = (wc >= cx1) & (wc < cx2) & (rr >= cy1) & (rr < cy2)
    masks_ref[...] = sg * cm.astype(f32)
    dets_ref[...] = dets
    cls_ref[...] = clsv


def kernel(bbox_pred_0, bbox_pred_1, bbox_pred_2, bbox_pred_3, bbox_pred_4,
           cls_score_0, cls_score_1, cls_score_2, cls_score_3, cls_score_4,
           coeff_pred_0, coeff_pred_1, coeff_pred_2, coeff_pred_3, coeff_pred_4,
           proto):
    bbox_preds = [bbox_pred_0, bbox_pred_1, bbox_pred_2, bbox_pred_3,
                  bbox_pred_4]
    cls_scores = [cls_score_0, cls_score_1, cls_score_2, cls_score_3,
                  cls_score_4]
    coeff_preds = [coeff_pred_0, coeff_pred_1, coeff_pred_2, coeff_pred_3,
                   coeff_pred_4]

    def prep(xs, ch):
        rows = [jnp.transpose(x[0], (1, 2, 0)).reshape(-1, ch) for x in xs]
        cat = jnp.concatenate(rows, axis=0)
        return jnp.pad(cat, ((0, _NP - _N), (0, 0))).T

    lT = prep(cls_scores, 81)
    dT = prep(bbox_preds, 4)
    cT = prep(coeff_preds, 32)
    aT = jnp.asarray(_ANCHORS)
    p2 = proto[0].reshape(6400, 32)

    dets, clsv, masks = pl.pallas_call(
        _yolact_kernel,
        out_shape=(
            jax.ShapeDtypeStruct((8, 128), jnp.float32),
            jax.ShapeDtypeStruct((8, 128), jnp.int32),
            jax.ShapeDtypeStruct((6400, 128), jnp.float32),
        ),
    )(lT, dT, cT, aT, p2)

    cls_dets = dets[:5, :_M].T
    classes = clsv[0, :_M]
    masks_o = masks[:, :_M].reshape(80, 80, _M)
    return cls_dets, classes, masks_o


# speedup vs baseline: 1.1294x; 1.1294x over previous
import numpy as np
import jax
import jax.numpy as jnp
from jax import lax
from jax.experimental import pallas as pl

_N = 6402
_NP = 6528
_C = 80
_K = 200
_M = 100
_FEATS = [40, 20, 10, 5, 3]
_STRIDES = [8, 16, 32, 64, 128]
_MAXR = 4.135166556742356


def _anchors_const():
    ratios = np.array([0.5, 1.0, 2.0], dtype=np.float64)
    h_r = np.sqrt(ratios)
    w_r = 1.0 / h_r
    out = []
    for bs, f, st in zip(_STRIDES, _FEATS, _STRIDES):
        cx = bs / 2.0
        cy = bs / 2.0
        ws = (bs * w_r * 3.0).astype(np.float32)
        hs = (bs * h_r * 3.0).astype(np.float32)
        base = np.stack([cx - 0.5 * ws, cy - 0.5 * hs,
                         cx + 0.5 * ws, cy + 0.5 * hs], axis=-1)
        sx = np.arange(f, dtype=np.float32) * st
        xx = np.tile(sx, f)
        yy = np.repeat(sx, f)
        shifts = np.stack([xx, yy, xx, yy], axis=-1)
        out.append((base[None, :, :] + shifts[:, None, :]).reshape(-1, 4))
    a = np.concatenate(out, axis=0).astype(np.float32)
    a = np.pad(a, ((0, _NP - _N), (0, 0)))
    return a.T.copy()


_ANCHORS = _anchors_const()


def _yolact_kernel(lg_ref, dl_ref, cf_ref, an_ref, pr_ref,
                   dets_ref, cls_ref, masks_ref):
    f32 = jnp.float32
    i32 = jnp.int32
    L = lg_ref[...]
    D = dl_ref[...]
    CO = cf_ref[...]
    A = an_ref[...]

    mx = jnp.max(L, axis=0, keepdims=True)
    e = jnp.exp(L - mx)
    p = e / jnp.sum(e, axis=0, keepdims=True)
    colN = lax.broadcasted_iota(i32, (1, _NP), 1)
    S = jnp.where(colN < _N, p[:_C, :], -1.0)

    ax1, ay1, ax2, ay2 = A[0:1], A[1:2], A[2:3], A[3:4]
    dx = D[0:1] * 0.1
    dy = D[1:2] * 0.1
    dw = jnp.clip(D[2:3] * 0.2, -_MAXR, _MAXR)
    dh = jnp.clip(D[3:4] * 0.2, -_MAXR, _MAXR)
    px = (ax1 + ax2) * 0.5
    py = (ay1 + ay2) * 0.5
    pw = ax2 - ax1
    ph = ay2 - ay1
    gx = px + pw * dx
    gy = py + ph * dy
    gw = pw * jnp.exp(dw)
    gh = ph * jnp.exp(dh)
    X1 = jnp.clip(gx - gw * 0.5, 0.0, 320.0)
    Y1 = jnp.clip(gy - gh * 0.5, 0.0, 320.0)
    X2 = jnp.clip(gx + gw * 0.5, 0.0, 320.0)
    Y2 = jnp.clip(gy + gh * 0.5, 0.0, 320.0)

    colCN = lax.broadcasted_iota(i32, (_C, _NP), 1)
    lane200 = lax.broadcasted_iota(i32, (1, _K), 1)

    def bcond(carry):
        return (carry[0] < _K) & (carry[-1] > 0.05)

    def bstep(carry):
        t, S_, tS, tI, _ = carry
        m = jnp.max(S_, axis=1, keepdims=True)
        am = jnp.min(jnp.where(S_ == m, colCN, _NP), axis=1,
                     keepdims=True)
        sel = lane200 == t
        tS = jnp.where(sel, m, tS)
        tI = jnp.where(sel, am, tI)
        S_ = jnp.where(colCN == am, -2.0, S_)
        return t + 1, S_, tS, tI, jnp.max(m)

    _, _, topS, topI, _ = lax.while_loop(
        bcond, bstep,
        (jnp.int32(0), S, jnp.zeros((_C, _K), f32),
         jnp.zeros((_C, _K), i32), jnp.float32(1.0)))

    iotaKN = lax.broadcasted_iota(i32, (_K, _NP), 1)
    rlt = (lax.broadcasted_iota(i32, (_K, _K), 0) <
           lax.broadcasted_iota(i32, (_K, _K), 1))
    rows80 = lax.broadcasted_iota(i32, (_C, 1), 0)

    def dstep(c, Mc):
        rsel = rows80 == c
        idx_c = jnp.sum(jnp.where(rsel, topI, 0), axis=0,
                        keepdims=True).reshape(_K, 1)
        cmask = iotaKN == idx_c
        x1 = jnp.sum(jnp.where(cmask, X1, 0.0), axis=1, keepdims=True)
        y1 = jnp.sum(jnp.where(cmask, Y1, 0.0), axis=1, keepdims=True)
        x2 = jnp.sum(jnp.where(cmask, X2, 0.0), axis=1, keepdims=True)
        y2 = jnp.sum(jnp.where(cmask, Y2, 0.0), axis=1, keepdims=True)
        x1t = x1.reshape(1, _K)
        y1t = y1.reshape(1, _K)
        x2t = x2.reshape(1, _K)
        y2t = y2.reshape(1, _K)
        w = jnp.maximum(jnp.minimum(x2, x2t) - jnp.maximum(x1, x1t), 0.0)
        h = jnp.maximum(jnp.minimum(y2, y2t) - jnp.maximum(y1, y1t), 0.0)
        ov = w * h
        ar = (x2 - x1) * (y2 - y1)
        un = jnp.maximum(ar + ar.reshape(1, _K) - ov, 1e-6)
        iou = ov / un
        im = jnp.max(jnp.where(rlt, iou, 0.0), axis=0, keepdims=True)
        s_c = jnp.sum(jnp.where(rsel, topS, 0.0), axis=0, keepdims=True)
        keep = (im <= 0.5) & (s_c > 0.05)
        mk = jnp.where(keep, s_c, -1.0)
        return jnp.where(rows80 == c, mk, Mc)

    Mmat = lax.fori_loop(0, _C, dstep, jnp.full((_C, _K), -1.0, f32))

    flatKC = (lax.broadcasted_iota(i32, (_C, _K), 0) * _K +
              lax.broadcasted_iota(i32, (_C, _K), 1))
    lane128 = lax.broadcasted_iota(i32, (1, 128), 1)
    row8 = lax.broadcasted_iota(i32, (8, 1), 0)

    def estep(k, carry):
        dets, clsv, CF, Mc = carry
        m = jnp.max(Mc)
        fl = jnp.min(jnp.where(Mc == m, flatKC, 1 << 30))
        cc = fl // _K
        anc = jnp.sum(jnp.where(flatKC == fl, topI, 0))
        valid = m > 0.0
        sc = jnp.where(valid, m, 0.0)
        amask = (colN == anc) & valid
        amf = amask.astype(f32)
        bx1 = jnp.sum(amf * X1)
        by1 = jnp.sum(amf * Y1)
        bx2 = jnp.sum(amf * X2)
        by2 = jnp.sum(amf * Y2)
        ck = jnp.sum(jnp.where(amask, CO, 0.0), axis=1, keepdims=True)
        sel = lane128 == k
        dets = jnp.where(sel & (row8 == 0), bx1, dets)
        dets = jnp.where(sel & (row8 == 1), by1, dets)
        dets = jnp.where(sel & (row8 == 2), bx2, dets)
        dets = jnp.where(sel & (row8 == 3), by2, dets)
        dets = jnp.where(sel & (row8 == 4), sc, dets)
        clsv = jnp.where(sel & (row8 == 0), cc, clsv)
        CF = jnp.where(sel, ck, CF)
        Mc = jnp.where(flatKC == fl, -2.0, Mc)
        return dets, clsv, CF, Mc

    dets, clsv, CF, _ = lax.fori_loop(
        0, _M, estep,
        (jnp.zeros((8, 128), f32), jnp.zeros((8, 128), i32),
         jnp.zeros((32, 128), f32), Mmat))

    P = pr_ref[...]
    mp = jnp.dot(P, CF, preferred_element_type=f32)
    sg = jax.nn.sigmoid(mp)
    pix = lax.broadcasted_iota(i32, (6400, 1), 0)
    rr = (pix // 80).astype(f32)
    wc = (pix % 80).astype(f32)
    cx1 = dets[0:1] * 0.25
    cy1 = dets[1:2] * 0.25
    cx2 = dets[2:3] * 0.25
    cy2 = dets[3:4] * 0.25
    cm = (wc >= cx1) & (wc < cx2) & (rr >= cy1) & (rr < cy2)
    masks_ref[...] = sg * cm.astype(f32)
    dets_ref[...] = dets
    cls_ref[...] = clsv


def kernel(bbox_pred_0, bbox_pred_1, bbox_pred_2, bbox_pred_3, bbox_pred_4,
           cls_score_0, cls_score_1, cls_score_2, cls_score_3, cls_score_4,
           coeff_pred_0, coeff_pred_1, coeff_pred_2, coeff_pred_3, coeff_pred_4,
           proto):
    bbox_preds = [bbox_pred_0, bbox_pred_1, bbox_pred_2, bbox_pred_3,
                  bbox_pred_4]
    cls_scores = [cls_score_0, cls_score_1, cls_score_2, cls_score_3,
                  cls_score_4]
    coeff_preds = [coeff_pred_0, coeff_pred_1, coeff_pred_2, coeff_pred_3,
                   coeff_pred_4]

    def prep(xs, ch):
        rows = [jnp.transpose(x[0], (1, 2, 0)).reshape(-1, ch) for x in xs]
        cat = jnp.concatenate(rows, axis=0)
        return jnp.pad(cat, ((0, _NP - _N), (0, 0))).T

    lT = prep(cls_scores, 81)
    dT = prep(bbox_preds, 4)
    cT = prep(coeff_preds, 32)
    aT = jnp.asarray(_ANCHORS)
    p2 = proto[0].reshape(6400, 32)

    dets, clsv, masks = pl.pallas_call(
        _yolact_kernel,
        out_shape=(
            jax.ShapeDtypeStruct((8, 128), jnp.float32),
            jax.ShapeDtypeStruct((8, 128), jnp.int32),
            jax.ShapeDtypeStruct((6400, 128), jnp.float32),
        ),
    )(lT, dT, cT, aT, p2)

    cls_dets = dets[:5, :_M].T
    classes = clsv[0, :_M]
    masks_o = masks[:, :_M].reshape(80, 80, _M)
    return cls_dets, classes, masks_o
